# Initial kernel scaffold; baseline (speedup 1.0000x reference)
import jax
import jax.numpy as jnp
from jax.experimental import pallas as pl


def kernel(est_source):
    raise NotImplementedError("write your pallas kernel here")


# speedup vs baseline: 1.6445x; 1.6445x over previous
import functools

import jax
import jax.numpy as jnp
from jax import lax
from jax.experimental import pallas as pl
from jax.experimental.pallas import tpu as pltpu
from jax.experimental.pallas import tpu_sc as plsc

NBC = 16
NROWS = 1600
NCOLS = 1000
HALF_ROWS = 800
CHUNK_ROWS = 20
NCHUNKS = 63
RW = 1008
HALF_OUT = 10240


def _src_col(k):
    return 16 * k if k < 62 else 984


def _dst_col(k):
    return 16 * k if k < 62 else 992


_GROUPS = ((0, 16), (16, 32), (32, 48), (48, NCHUNKS))

_mesh = plsc.VectorSubcoreMesh(core_axis_name="c", subcore_axis_name="s")


@functools.partial(
    pl.kernel,
    mesh=_mesh,
    compiler_params=pltpu.CompilerParams(
        use_tc_tiling_on_sc=False, needs_layout_passes=False),
    out_type=(
        jax.ShapeDtypeStruct((NBC, 8, 1280), jnp.float32),
        jax.ShapeDtypeStruct((NBC, 8, 1280), jnp.float32),
    ),
    scratch_types=[
        pltpu.VMEM((CHUNK_ROWS, NCOLS), jnp.float32),
        pltpu.VMEM((CHUNK_ROWS, NCOLS), jnp.float32),
        pltpu.VMEM((20, RW), jnp.float32),
        pltpu.VMEM((20, RW), jnp.float32),
        pltpu.VMEM((8, 1280), jnp.float32),
        pltpu.VMEM_SHARED((8, 2, 20, RW), jnp.float32),
        pltpu.SemaphoreType.DMA,
        pltpu.SemaphoreType.DMA,
    ],
)
def _decoder_sc(x_hbm, out_a_hbm, out_b_hbm, in0, in1, r_my, r_other, out_v,
                spmem, sem0, sem1):
    c_idx = lax.axis_index("c")
    s_idx = lax.axis_index("s")
    pair = s_idx // 2
    j = s_idx % 2
    bc = c_idx * 8 + pair
    row_base = j * HALF_ROWS

    def dma(blk, buf, sem):
        return pltpu.make_async_copy(
            x_hbm.at[bc, pl.ds(row_base + blk * CHUNK_ROWS, CHUNK_ROWS), :],
            buf, sem)

    def accumulate(blk, buf, first):
        for g0, g1 in _GROUPS:
            nk = g1 - g0

            def r_body(r, acc, _g0=g0, _nk=nk):
                return tuple(
                    acc[i] + buf[r, pl.ds(_src_col(_g0 + i), 16)]
                    for i in range(_nk))

            acc = lax.fori_loop(
                0, CHUNK_ROWS, r_body,
                tuple(jnp.zeros((16,), jnp.float32) for _ in range(nk)))
            for i in range(nk):
                dst = _dst_col(g0 + i)
                if first:
                    r_my[blk, pl.ds(dst, 16)] = acc[i]
                else:
                    plsc.addupdate(r_my.at[blk, pl.ds(dst, 16)], acc[i])

    dma(0, in0, sem0).start()
    dma(1, in1, sem1).start()

    def m_body(m, carry):
        dma(2 * m, in0, sem0).wait()
        accumulate(m, in0, first=True)

        @pl.when(m < 19)
        def _():
            dma(2 * m + 2, in0, sem0).start()

        dma(2 * m + 1, in1, sem1).wait()
        accumulate(m, in1, first=False)

        @pl.when(m < 19)
        def _():
            dma(2 * m + 3, in1, sem1).start()

        return carry

    lax.fori_loop(0, 20, m_body, 0)

    pltpu.sync_copy(r_my, spmem.at[pair, j])
    plsc.subcore_barrier()
    pltpu.sync_copy(spmem.at[pair, 1 - j], r_other)

    zero = jnp.zeros((16,), jnp.float32)
    iot = lax.iota(jnp.int32, 16)

    def epilogue(half_a, half_b, t0):
        def rr_body(rr, carry):
            def ii_body(ii, carry2):
                t = t0 + (rr * 80 + ii) * 16 + iot
                tf = t.astype(jnp.float32)
                s = (tf * jnp.float32(0.05)
                     + jnp.float32(1e-3)).astype(jnp.int32)
                u = t - 20 * s
                col1 = jnp.minimum(s, 999)
                col1 = jnp.where(col1 >= 984, col1 + 8, col1)
                g1 = plsc.load_gather(half_a, [u, col1])
                v1 = jnp.where(s <= 999, g1, zero)
                col2 = jnp.minimum(jnp.maximum(s - 1, 0), 999)
                col2 = jnp.where(col2 >= 984, col2 + 8, col2)
                g2 = plsc.load_gather(half_b, [u, col2])
                v2 = jnp.where((s >= 1) & (s <= 1000), g2, zero)
                out_v[rr, pl.ds(16 * ii, 16)] = (v1 + v2) * jnp.float32(0.025)
                return carry2

            lax.fori_loop(0, 80, ii_body, 0)
            return carry

        lax.fori_loop(0, 8, rr_body, 0)

    @pl.when(j == 0)
    def _():
        epilogue(r_my, r_other, 0)
        pltpu.sync_copy(out_v, out_a_hbm.at[bc])

    @pl.when(j == 1)
    def _():
        epilogue(r_other, r_my, HALF_OUT)
        pltpu.sync_copy(out_v, out_b_hbm.at[bc])


@jax.jit
def kernel(est_source):
    x = est_source.reshape(NBC, NROWS, NCOLS)
    out_a, out_b = _decoder_sc(x)
    full = jnp.concatenate(
        [out_a.reshape(NBC, HALF_OUT), out_b.reshape(NBC, HALF_OUT)], axis=1)
    return full[:, :20020].reshape(8, 2, 20020)


# speedup vs baseline: 3.6636x; 2.2278x over previous
import functools

import jax
import jax.numpy as jnp
from jax import lax
from jax.experimental import pallas as pl
from jax.experimental.pallas import tpu as pltpu
from jax.experimental.pallas import tpu_sc as plsc

NBC = 16
NROWS = 1600
NCOLS = 1000
HALF_ROWS = 800
CHUNK_ROWS = 40
NCHUNKS = 63
RW = 1024
HALF_OUT = 10240


def _src_col(k):
    return 16 * k if k < 62 else 984


def _dst_col(k):
    return 16 * k if k < 62 else 992


_GROUPS = ((0, 16), (16, 32), (32, 48), (48, NCHUNKS))

_mesh = plsc.VectorSubcoreMesh(core_axis_name="c", subcore_axis_name="s")


def _worker(ctx=None):
    c_idx = lax.axis_index("c")
    s_idx = lax.axis_index("s")
    pair = s_idx // 2
    j = s_idx % 2
    bc = c_idx * 8 + pair
    return bc, j


@functools.partial(
    pl.kernel,
    mesh=_mesh,
    out_type=jax.ShapeDtypeStruct((NBC, 2, 24, RW), jnp.float32),
    scratch_types=[
        pltpu.VMEM((CHUNK_ROWS, NCOLS), jnp.float32),
        pltpu.VMEM((CHUNK_ROWS, NCOLS), jnp.float32),
        pltpu.VMEM((24, RW), jnp.float32),
        pltpu.SemaphoreType.DMA,
        pltpu.SemaphoreType.DMA,
    ],
)
def _reduce_sc(x_hbm, r_hbm, in0, in1, r_my, sem0, sem1):
    bc, j = _worker()
    row_base = j * HALF_ROWS

    def dma(blk, buf, sem):
        return pltpu.make_async_copy(
            x_hbm.at[bc, pl.ds(row_base + blk * CHUNK_ROWS, CHUNK_ROWS), :],
            buf, sem)

    def accumulate(blk, buf):
        for g0, g1 in _GROUPS:
            nk = g1 - g0

            def r_body(r, acc, _g0=g0, _nk=nk):
                return tuple(
                    acc[i] + buf[r, pl.ds(_src_col(_g0 + i), 16)]
                    for i in range(_nk))

            acc = lax.fori_loop(
                0, CHUNK_ROWS, r_body,
                tuple(jnp.zeros((16,), jnp.float32) for _ in range(nk)))
            for i in range(nk):
                r_my[blk, pl.ds(_dst_col(g0 + i), 16)] = acc[i]

    dma(0, in0, sem0).start()
    dma(1, in1, sem1).start()

    def m_body(m, carry):
        dma(2 * m, in0, sem0).wait()
        accumulate(2 * m, in0)

        @pl.when(m < 9)
        def _():
            dma(2 * m + 2, in0, sem0).start()

        dma(2 * m + 1, in1, sem1).wait()
        accumulate(2 * m + 1, in1)

        @pl.when(m < 9)
        def _():
            dma(2 * m + 3, in1, sem1).start()

        return carry

    lax.fori_loop(0, 10, m_body, 0)

    pltpu.sync_copy(r_my, r_hbm.at[bc, j])


@functools.partial(
    pl.kernel,
    mesh=_mesh,
    compiler_params=pltpu.CompilerParams(
        use_tc_tiling_on_sc=False, needs_layout_passes=False),
    out_type=(
        jax.ShapeDtypeStruct((NBC, 8, 1280), jnp.float32),
        jax.ShapeDtypeStruct((NBC, 8, 1280), jnp.float32),
    ),
    scratch_types=[
        pltpu.VMEM((20, RW), jnp.float32),
        pltpu.VMEM((20, RW), jnp.float32),
        pltpu.VMEM((8, 1280), jnp.float32),
    ],
)
def _stitch_sc(r_hbm, out_a_hbm, out_b_hbm, r_a, r_b, out_v):
    bc, j = _worker()

    pltpu.sync_copy(r_hbm.at[bc, 0, pl.ds(0, 20), :], r_a)
    pltpu.sync_copy(r_hbm.at[bc, 1, pl.ds(0, 20), :], r_b)

    zero = jnp.zeros((16,), jnp.float32)
    iot = lax.iota(jnp.int32, 16)

    def epilogue(t0):
        def rr_body(rr, carry):
            def ii_body(ii, carry2):
                t = t0 + (rr * 80 + ii) * 16 + iot
                tf = t.astype(jnp.float32)
                s = (tf * jnp.float32(0.05)
                     + jnp.float32(1e-3)).astype(jnp.int32)
                u = t - 20 * s
                col1 = jnp.minimum(s, 999)
                col1 = jnp.where(col1 >= 984, col1 + 8, col1)
                g1 = plsc.load_gather(r_a, [u, col1])
                v1 = jnp.where(s <= 999, g1, zero)
                col2 = jnp.minimum(jnp.maximum(s - 1, 0), 999)
                col2 = jnp.where(col2 >= 984, col2 + 8, col2)
                g2 = plsc.load_gather(r_b, [u, col2])
                v2 = jnp.where((s >= 1) & (s <= 1000), g2, zero)
                out_v[rr, pl.ds(16 * ii, 16)] = (v1 + v2) * jnp.float32(0.025)
                return carry2

            lax.fori_loop(0, 80, ii_body, 0)
            return carry

        lax.fori_loop(0, 8, rr_body, 0)

    @pl.when(j == 0)
    def _():
        epilogue(0)
        pltpu.sync_copy(out_v, out_a_hbm.at[bc])

    @pl.when(j == 1)
    def _():
        epilogue(HALF_OUT)
        pltpu.sync_copy(out_v, out_b_hbm.at[bc])


@jax.jit
def kernel(est_source):
    x = est_source.reshape(NBC, NROWS, NCOLS)
    r = _reduce_sc(x)
    out_a, out_b = _stitch_sc(r)
    full = jnp.concatenate(
        [out_a.reshape(NBC, HALF_OUT), out_b.reshape(NBC, HALF_OUT)], axis=1)
    return full[:, :20020].reshape(8, 2, 20020)


# speedup vs baseline: 3.7891x; 1.0343x over previous
import functools

import jax
import jax.numpy as jnp
from jax import lax
from jax.experimental import pallas as pl
from jax.experimental.pallas import tpu as pltpu
from jax.experimental.pallas import tpu_sc as plsc

NBC = 16
NROWS = 1600
NCOLS = 1000
HALF_ROWS = 800
CHUNK_ROWS = 40
NCHUNKS = 63
RW = 1024
RHALF = 24 * RW
HALF_OUT = 10240


def _col0(k):
    return 16 * k if k < 62 else 984


_GROUPS = ((0, 16), (16, 32), (32, 48), (48, NCHUNKS))

_mesh = plsc.VectorSubcoreMesh(core_axis_name="c", subcore_axis_name="s")


def _worker():
    c_idx = lax.axis_index("c")
    s_idx = lax.axis_index("s")
    pair = s_idx // 2
    j = s_idx % 2
    bc = c_idx * 8 + pair
    return bc, j


@functools.partial(
    pl.kernel,
    mesh=_mesh,
    out_type=jax.ShapeDtypeStruct((NBC * 2 * RHALF,), jnp.float32),
    scratch_types=[
        pltpu.VMEM((CHUNK_ROWS, NCOLS), jnp.float32),
        pltpu.VMEM((CHUNK_ROWS, NCOLS), jnp.float32),
        pltpu.VMEM((RHALF,), jnp.float32),
        pltpu.SemaphoreType.DMA,
        pltpu.SemaphoreType.DMA,
    ],
)
def _reduce_sc(x_hbm, r_hbm, in0, in1, r_my, sem0, sem1):
    bc, j = _worker()
    row_base = j * HALF_ROWS

    def dma(blk, buf, sem):
        return pltpu.make_async_copy(
            x_hbm.at[bc, pl.ds(row_base + blk * CHUNK_ROWS, CHUNK_ROWS), :],
            buf, sem)

    def accumulate(blk, buf):
        for g0, g1 in _GROUPS:
            nk = g1 - g0

            def r_body(r2, acc, _g0=g0, _nk=nk):
                r = 2 * r2
                acc = tuple(
                    acc[i] + buf[r, pl.ds(_col0(_g0 + i), 16)]
                    for i in range(_nk))
                return tuple(
                    acc[i] + buf[r + 1, pl.ds(_col0(_g0 + i), 16)]
                    for i in range(_nk))

            acc = lax.fori_loop(
                0, CHUNK_ROWS // 2, r_body,
                tuple(jnp.zeros((16,), jnp.float32) for _ in range(nk)))
            for i in range(nk):
                r_my[pl.ds(blk * RW + _col0(g0 + i), 16)] = acc[i]

    dma(0, in0, sem0).start()
    dma(1, in1, sem1).start()

    def m_body(m, carry):
        dma(2 * m, in0, sem0).wait()
        accumulate(2 * m, in0)

        @pl.when(m < 9)
        def _():
            dma(2 * m + 2, in0, sem0).start()

        dma(2 * m + 1, in1, sem1).wait()
        accumulate(2 * m + 1, in1)

        @pl.when(m < 9)
        def _():
            dma(2 * m + 3, in1, sem1).start()

        return carry

    lax.fori_loop(0, 10, m_body, 0)

    pltpu.sync_copy(r_my, r_hbm.at[pl.ds((bc * 2 + j) * RHALF, RHALF)])


@functools.partial(
    pl.kernel,
    mesh=_mesh,
    compiler_params=pltpu.CompilerParams(
        use_tc_tiling_on_sc=False, needs_layout_passes=False),
    out_type=(
        jax.ShapeDtypeStruct((NBC, 8, 1280), jnp.float32),
        jax.ShapeDtypeStruct((NBC, 8, 1280), jnp.float32),
    ),
    scratch_types=[
        pltpu.VMEM((20 * RW,), jnp.float32),
        pltpu.VMEM((20 * RW,), jnp.float32),
        pltpu.VMEM((8, 1280), jnp.float32),
    ],
)
def _stitch_sc(r_hbm, out_a_hbm, out_b_hbm, r_a, r_b, out_v):
    bc, j = _worker()

    pltpu.sync_copy(r_hbm.at[pl.ds((bc * 2) * RHALF, 20 * RW)], r_a)
    pltpu.sync_copy(r_hbm.at[pl.ds((bc * 2 + 1) * RHALF, 20 * RW)], r_b)

    zero = jnp.zeros((16,), jnp.float32)
    iot = lax.iota(jnp.int32, 16)

    def epilogue(t0):
        def rr_body(rr, carry):
            def ii_body(ii, carry2):
                t = t0 + (rr * 80 + ii) * 16 + iot
                tf = t.astype(jnp.float32)
                s = (tf * jnp.float32(0.05)
                     + jnp.float32(1e-3)).astype(jnp.int32)
                u = t - 20 * s
                u_row = u * RW
                col1 = jnp.minimum(s, 999)
                g1 = plsc.load_gather(r_a, [u_row + col1])
                v1 = jnp.where(s <= 999, g1, zero)
                col2 = jnp.minimum(jnp.maximum(s - 1, 0), 999)
                g2 = plsc.load_gather(r_b, [u_row + col2])
                v2 = jnp.where((s >= 1) & (s <= 1000), g2, zero)
                out_v[rr, pl.ds(16 * ii, 16)] = (v1 + v2) * jnp.float32(0.025)
                return carry2

            lax.fori_loop(0, 80, ii_body, 0)
            return carry

        lax.fori_loop(0, 8, rr_body, 0)

    @pl.when(j == 0)
    def _():
        epilogue(0)
        pltpu.sync_copy(out_v, out_a_hbm.at[bc])

    @pl.when(j == 1)
    def _():
        epilogue(HALF_OUT)
        pltpu.sync_copy(out_v, out_b_hbm.at[bc])


@jax.jit
def kernel(est_source):
    x = est_source.reshape(NBC, NROWS, NCOLS)
    r = _reduce_sc(x)
    out_a, out_b = _stitch_sc(r)
    full = jnp.concatenate(
        [out_a.reshape(NBC, HALF_OUT), out_b.reshape(NBC, HALF_OUT)], axis=1)
    return full[:, :20020].reshape(8, 2, 20020)
